# baseline (device time: 406766 ns/iter reference)
import jax
import jax.numpy as jnp
from jax import lax
from jax.experimental import pallas as pl
from jax.experimental.pallas import tpu as pltpu

Bb, S, D, N = 8, 512, 512, 16
DH = D // 2
NCHUNK = 8
TC = S // NCHUNK


def kernel(x, A, B, C):
    my_x = lax.axis_index("x")
    xh = lax.dynamic_slice_in_dim(x, my_x * DH, DH, axis=2)
    dAT = jnp.exp(A).T
    dATh = lax.dynamic_slice_in_dim(dAT, my_x * DH, DH, axis=1)

    def body(xh_ref, dATh_ref, B_ref, C_ref, mine_ref, theirs_ref, h_ref,
             h_send_sem, h_recv_sem, ack_sem, out_send_sems, out_recv_sems):
        mx = lax.axis_index("x")
        my = lax.axis_index("y")
        other_x = 1 - mx

        h_rdma = pltpu.make_async_remote_copy(
            src_ref=h_ref,
            dst_ref=h_ref,
            send_sem=h_send_sem,
            recv_sem=h_recv_sem,
            device_id=(mx, 1),
            device_id_type=pl.DeviceIdType.MESH,
        )

        @pl.when(my == 0)
        def _():
            h_ref[...] = jnp.zeros_like(h_ref)

        @pl.when(my == 1)
        def _():
            h_rdma.wait_recv()
            pl.semaphore_signal(
                ack_sem, inc=1,
                device_id=(mx, 0),
                device_id_type=pl.DeviceIdType.MESH,
            )

        def out_chunk_rdma(c):
            return pltpu.make_async_remote_copy(
                src_ref=mine_ref.at[:, c * TC:(c + 1) * TC, :],
                dst_ref=theirs_ref.at[:, c * TC:(c + 1) * TC, :],
                send_sem=out_send_sems.at[c],
                recv_sem=out_recv_sems.at[c],
                device_id=(other_x, my),
                device_id_type=pl.DeviceIdType.MESH,
            )

        for c in range(NCHUNK):
            def step(t, carry):
                gt = c * TC + t
                x_t = xh_ref[:, gt, :]
                y_t = jnp.zeros_like(x_t)
                for n in range(N):
                    dA_n = dATh_ref[n:n + 1, :]
                    b_tn = B_ref[:, gt, n:n + 1]
                    c_tn = C_ref[:, gt, n:n + 1]
                    h_n = h_ref[n] * dA_n + x_t * b_tn
                    h_ref[n] = h_n
                    y_t = y_t + h_n * c_tn
                mine_ref[:, gt, :] = y_t
                return carry

            lax.fori_loop(0, TC, step, 0)
            out_chunk_rdma(c).start()

        @pl.when(my == 0)
        def _():
            h_rdma.start()

        for c in range(NCHUNK):
            d = out_chunk_rdma(c)
            d.wait_send()
            d.wait_recv()

        @pl.when(my == 0)
        def _():
            h_rdma.wait_send()
            pl.semaphore_wait(ack_sem, 1)

    mine, theirs = pl.pallas_call(
        body,
        out_shape=(
            jax.ShapeDtypeStruct((Bb, S, DH), jnp.float32),
            jax.ShapeDtypeStruct((Bb, S, DH), jnp.float32),
        ),
        in_specs=[pl.BlockSpec(memory_space=pltpu.VMEM)] * 4,
        out_specs=(
            pl.BlockSpec(memory_space=pltpu.VMEM),
            pl.BlockSpec(memory_space=pltpu.VMEM),
        ),
        scratch_shapes=[
            pltpu.VMEM((N, Bb, DH), jnp.float32),
            pltpu.SemaphoreType.DMA,
            pltpu.SemaphoreType.DMA,
            pltpu.SemaphoreType.REGULAR,
            pltpu.SemaphoreType.DMA((NCHUNK,)),
            pltpu.SemaphoreType.DMA((NCHUNK,)),
        ],
    )(xh, dATh, B, C)

    lo = jnp.concatenate([mine, theirs], axis=-1)
    hi = jnp.concatenate([theirs, mine], axis=-1)
    return jnp.where(my_x == 0, lo, hi)


# device time: 67060 ns/iter; 6.0657x vs baseline; 6.0657x over previous
import jax
import jax.numpy as jnp
from jax import lax
from jax.experimental import pallas as pl
from jax.experimental.pallas import tpu as pltpu

Bb, S, D, N = 8, 512, 512, 16
DH = D // 2
TC = 16
K = S // TC


def kernel(x, A, B, C):
    my_x = lax.axis_index("x")
    xh = lax.dynamic_slice_in_dim(x, my_x * DH, DH, axis=2)
    x4 = xh.reshape(Bb, K, TC, DH).astype(jnp.bfloat16)
    Bp = B.reshape(Bb, K, TC, N).transpose(0, 1, 3, 2).astype(jnp.bfloat16)
    Cp = C.reshape(Bb, K, TC, N).transpose(0, 1, 3, 2).astype(jnp.bfloat16)
    dAT = jnp.exp(A).T
    dAT16 = jnp.exp(16.0 * A).T
    dATh = lax.dynamic_slice_in_dim(dAT, my_x * DH, DH, axis=1)
    dAT16h = lax.dynamic_slice_in_dim(dAT16, my_x * DH, DH, axis=1)

    def body(x4_ref, Bp_ref, Cp_ref, dATh_ref, dAT16h_ref,
             mine_ref, theirs_ref, rg_ref, h_ref,
             h_send_sem, h_recv_sem, ack_sem, out_send_sems, out_recv_sems):
        mx = lax.axis_index("x")
        my = lax.axis_index("y")
        other_x = 1 - mx

        h_rdma = pltpu.make_async_remote_copy(
            src_ref=h_ref,
            dst_ref=h_ref,
            send_sem=h_send_sem,
            recv_sem=h_recv_sem,
            device_id=(mx, 1),
            device_id_type=pl.DeviceIdType.MESH,
        )

        @pl.when(my == 0)
        def _():
            h_ref[...] = jnp.zeros_like(h_ref)

        dAb = dATh_ref[...][None, None, :, :]
        dA16b = dAT16h_ref[...][None, :, :]

        def u_step(tau):
            xs = x4_ref[:, :, tau, :].astype(jnp.float32)
            bs = Bp_ref[:, :, :, tau].astype(jnp.float32)
            return xs[:, :, None, :] * bs[:, :, :, None]

        rg_ref[...] = u_step(0)
        for tau in range(1, TC):
            rg_ref[...] = rg_ref[...] * dAb + u_step(tau)

        @pl.when(my == 1)
        def _():
            h_rdma.wait_recv()
            pl.semaphore_signal(
                ack_sem, inc=1,
                device_id=(mx, 0),
                device_id_type=pl.DeviceIdType.MESH,
            )

        H = h_ref[...]
        for k in range(K):
            Rk = rg_ref[:, k]
            rg_ref[:, k] = H
            H = H * dA16b + Rk
        h_ref[...] = H

        @pl.when(my == 0)
        def _():
            h_rdma.start()

        def out_rdma(chunk):
            sl = pl.ds(chunk * 8, 8)
            return pltpu.make_async_remote_copy(
                src_ref=mine_ref.at[:, :, sl, :],
                dst_ref=theirs_ref.at[:, :, sl, :],
                send_sem=out_send_sems.at[chunk],
                recv_sem=out_recv_sems.at[chunk],
                device_id=(other_x, my),
                device_id_type=pl.DeviceIdType.MESH,
            )

        for tau in range(TC):
            g = rg_ref[...] * dAb + u_step(tau)
            rg_ref[...] = g
            cs = Cp_ref[:, :, :, tau].astype(jnp.float32)
            y4 = jnp.sum(g * cs[:, :, :, None], axis=2)
            mine_ref[:, :, tau, :] = y4.astype(jnp.bfloat16)
            if tau % 8 == 7:
                out_rdma(tau // 8).start()

        for chunk in range(TC // 8):
            d = out_rdma(chunk)
            d.wait_send()
            d.wait_recv()

        @pl.when(my == 0)
        def _():
            h_rdma.wait_send()
            pl.semaphore_wait(ack_sem, 1)

    mine, theirs = pl.pallas_call(
        body,
        out_shape=(
            jax.ShapeDtypeStruct((Bb, K, TC, DH), jnp.bfloat16),
            jax.ShapeDtypeStruct((Bb, K, TC, DH), jnp.bfloat16),
        ),
        in_specs=[pl.BlockSpec(memory_space=pltpu.VMEM)] * 5,
        out_specs=(
            pl.BlockSpec(memory_space=pltpu.VMEM),
            pl.BlockSpec(memory_space=pltpu.VMEM),
        ),
        scratch_shapes=[
            pltpu.VMEM((Bb, K, N, DH), jnp.float32),
            pltpu.VMEM((Bb, N, DH), jnp.float32),
            pltpu.SemaphoreType.DMA,
            pltpu.SemaphoreType.DMA,
            pltpu.SemaphoreType.REGULAR,
            pltpu.SemaphoreType.DMA((TC // 8,)),
            pltpu.SemaphoreType.DMA((TC // 8,)),
        ],
    )(x4, Bp, Cp, dATh, dAT16h)

    mine = mine.reshape(Bb, S, DH)
    theirs = theirs.reshape(Bb, S, DH)
    lo = jnp.concatenate([mine, theirs], axis=-1)
    hi = jnp.concatenate([theirs, mine], axis=-1)
    return jnp.where(my_x == 0, lo, hi)


# device time: 65452 ns/iter; 6.2147x vs baseline; 1.0246x over previous
import jax
import jax.numpy as jnp
from jax import lax
from jax.experimental import pallas as pl
from jax.experimental.pallas import tpu as pltpu

Bb, S, D, N = 8, 512, 512, 16
DH = D // 2
TC = 16
K = S // TC


def kernel(x, A, B, C):
    my_x = lax.axis_index("x")
    xh = lax.dynamic_slice_in_dim(x, my_x * DH, DH, axis=2)
    x4 = xh.reshape(Bb, K, TC, DH).astype(jnp.bfloat16)
    Bp = B.reshape(Bb, K, TC, N).transpose(0, 1, 3, 2).astype(jnp.bfloat16)
    Cp = C.reshape(Bb, K, TC, N).transpose(0, 1, 3, 2).astype(jnp.bfloat16)
    dAT = jnp.exp(A).T
    dAT16 = jnp.exp(16.0 * A).T
    dATh = lax.dynamic_slice_in_dim(dAT, my_x * DH, DH, axis=1)
    dAT16h = lax.dynamic_slice_in_dim(dAT16, my_x * DH, DH, axis=1)

    def body(x4_ref, Bp_ref, Cp_ref, dATh_ref, dAT16h_ref,
             mine_ref, theirs_ref, rg_ref, h_ref,
             h_send_sem, h_recv_sem, ack_sem, out_send_sems, out_recv_sems):
        mx = lax.axis_index("x")
        my = lax.axis_index("y")
        other_x = 1 - mx

        h_rdma = pltpu.make_async_remote_copy(
            src_ref=h_ref,
            dst_ref=h_ref,
            send_sem=h_send_sem,
            recv_sem=h_recv_sem,
            device_id=(mx, 1),
            device_id_type=pl.DeviceIdType.MESH,
        )

        @pl.when(my == 0)
        def _():
            h_ref[...] = jnp.zeros_like(h_ref)

        dAb = dATh_ref[...].astype(jnp.bfloat16)[None, None, :, :]
        dA16b = dAT16h_ref[...][None, :, :]

        def u_step(tau):
            xs = x4_ref[:, :, tau, :]
            bs = Bp_ref[:, :, :, tau]
            return xs[:, :, None, :] * bs[:, :, :, None]

        rg_ref[...] = u_step(0)
        for tau in range(1, TC):
            rg_ref[...] = rg_ref[...] * dAb + u_step(tau)

        @pl.when(my == 1)
        def _():
            h_rdma.wait_recv()
            pl.semaphore_signal(
                ack_sem, inc=1,
                device_id=(mx, 0),
                device_id_type=pl.DeviceIdType.MESH,
            )

        H = h_ref[...]
        for k in range(K):
            Rk = rg_ref[:, k].astype(jnp.float32)
            rg_ref[:, k] = H.astype(jnp.bfloat16)
            H = H * dA16b + Rk
        h_ref[...] = H

        @pl.when(my == 0)
        def _():
            h_rdma.start()

        def out_rdma(chunk):
            sl = pl.ds(chunk * 8, 8)
            return pltpu.make_async_remote_copy(
                src_ref=mine_ref.at[:, :, sl, :],
                dst_ref=theirs_ref.at[:, :, sl, :],
                send_sem=out_send_sems.at[chunk],
                recv_sem=out_recv_sems.at[chunk],
                device_id=(other_x, my),
                device_id_type=pl.DeviceIdType.MESH,
            )

        for tau in range(TC):
            g = rg_ref[...] * dAb + u_step(tau)
            rg_ref[...] = g
            cs = Cp_ref[:, :, :, tau]
            y4 = jnp.sum(g * cs[:, :, :, None], axis=2)
            mine_ref[:, :, tau, :] = y4
            if tau % 8 == 7:
                out_rdma(tau // 8).start()

        for chunk in range(TC // 8):
            d = out_rdma(chunk)
            d.wait_send()
            d.wait_recv()

        @pl.when(my == 0)
        def _():
            h_rdma.wait_send()
            pl.semaphore_wait(ack_sem, 1)

    mine, theirs = pl.pallas_call(
        body,
        out_shape=(
            jax.ShapeDtypeStruct((Bb, K, TC, DH), jnp.bfloat16),
            jax.ShapeDtypeStruct((Bb, K, TC, DH), jnp.bfloat16),
        ),
        in_specs=[pl.BlockSpec(memory_space=pltpu.VMEM)] * 5,
        out_specs=(
            pl.BlockSpec(memory_space=pltpu.VMEM),
            pl.BlockSpec(memory_space=pltpu.VMEM),
        ),
        scratch_shapes=[
            pltpu.VMEM((Bb, K, N, DH), jnp.bfloat16),
            pltpu.VMEM((Bb, N, DH), jnp.float32),
            pltpu.SemaphoreType.DMA,
            pltpu.SemaphoreType.DMA,
            pltpu.SemaphoreType.REGULAR,
            pltpu.SemaphoreType.DMA((TC // 8,)),
            pltpu.SemaphoreType.DMA((TC // 8,)),
        ],
    )(x4, Bp, Cp, dATh, dAT16h)

    mine = mine.reshape(Bb, S, DH)
    theirs = theirs.reshape(Bb, S, DH)
    lo = jnp.concatenate([mine, theirs], axis=-1)
    hi = jnp.concatenate([theirs, mine], axis=-1)
    return jnp.where(my_x == 0, lo, hi)
